# baseline (speedup 1.0000x reference)
import functools

import jax
import jax.numpy as jnp
from jax import lax
from jax.experimental import pallas as pl
from jax.experimental.pallas import tpu as pltpu
from jax.experimental.pallas import tpu_sc as plsc

NUM_BINS = 64
EMBED_DIM = 64

_NC = 2
_NS = 16
_NW = _NC * _NS

_N = 100000
_NPAD = 100096
_W_PER_ROW = 4
_STRIPE = 25088
_CHUNK = 256
_PAIRS = _CHUNK // 2
_TPITCH = _CHUNK + 1
_NPIPE = 96
_PAIRS_MAX = _STRIPE // 2
_PAIRS_T3 = (_N - 3 * _STRIPE) // 2


def _transpose_chunk(rows, tbuf, row_vecs):
    def pgroup(g, _):
        for q in range(16):
            p = g * 16 + q
            col_lo = lax.broadcast(2 * p, (16,))
            col_hi = col_lo + 1
            for ci in range(8):
                v = rows[p, pl.ds(ci * 16, 16)]
                plsc.store_scatter(
                    tbuf, [row_vecs[ci % 4], col_lo if ci < 4 else col_hi], v)
        return 0

    lax.fori_loop(0, _PAIRS // 16, pgroup, 0)


def _sc_embed(idx2_hbm, tab_hbm, out_hbm,
              idxall, rows0, rows1, tbuf0, tbuf1,
              sem_g0, sem_g1, sem_o0, sem_o1):
    c = lax.axis_index("c")
    s = lax.axis_index("s")
    wid = s * _NC + c
    out_b = wid // _W_PER_ROW
    t = wid % _W_PER_ROW
    out_n0 = t * _STRIPE
    pair_base = out_b * (_N // 2) + t * _PAIRS_MAX

    rows = (rows0, rows1)
    tbuf = (tbuf0, tbuf1)
    sem_g = (sem_g0, sem_g1)
    sem_o = (sem_o0, sem_o1)

    iota = lax.iota(jnp.int32, 16)
    row_vecs = [iota + 16 * ci for ci in range(4)]

    @pl.when(t < _W_PER_ROW - 1)
    def _():
        pltpu.async_copy(
            idx2_hbm.at[pl.ds(pair_base, _PAIRS_MAX)], idxall, sem_g0).wait()

    @pl.when(t == _W_PER_ROW - 1)
    def _():
        pltpu.async_copy(
            idx2_hbm.at[pl.ds(pair_base, _PAIRS_T3)],
            idxall.at[pl.ds(0, _PAIRS_T3)], sem_g0).wait()
        zeros = jnp.zeros((16,), jnp.int32)
        for z in range(_PAIRS_T3, (_NPIPE + 1) * _PAIRS, 16):
            idxall[pl.ds(z, 16)] = zeros

    def gather(j_pairs, sl):
        return pltpu.make_async_copy(
            tab_hbm.at[idxall.at[pl.ds(j_pairs, _PAIRS)]], rows[sl], sem_g[sl])

    def store(j, sl):
        return pltpu.make_async_copy(
            tbuf[sl].at[:, pl.ds(0, _CHUNK)],
            out_hbm.at[out_b, :, pl.ds(out_n0 + j * _CHUNK, _CHUNK)],
            sem_o[sl])

    gather(0, 0).start()
    gather(_PAIRS, 1).start()

    def step(k, _):
        for sl in range(2):
            j = 2 * k + sl
            jp = j * _PAIRS
            gather(jp, sl).wait()

            @pl.when(k > 0)
            def _():
                store(j - 2, sl).wait()

            _transpose_chunk(rows[sl], tbuf[sl], row_vecs)
            store(j, sl).start()

            @pl.when(k < _NPIPE // 2 - 1)
            def _():
                gather(jp + 2 * _PAIRS, sl).start()
        return 0

    lax.fori_loop(0, _NPIPE // 2, step, 0)
    for sl in range(2):
        store(_NPIPE - 2 + sl, sl).wait()

    g0 = gather(_NPIPE * _PAIRS, 0)
    g0.start()
    g0.wait()
    _transpose_chunk(rows0, tbuf0, row_vecs)
    s0 = store(_NPIPE, 0)
    s0.start()
    s0.wait()

    @pl.when(t < _W_PER_ROW - 1)
    def _():
        g1 = gather((_NPIPE + 1) * _PAIRS, 1)
        g1.start()
        g1.wait()
        _transpose_chunk(rows1, tbuf1, row_vecs)
        s1 = store(_NPIPE + 1, 1)
        s1.start()
        s1.wait()


@jax.jit
def _run(idx2, tab_pair):
    mesh = plsc.VectorSubcoreMesh(core_axis_name="c", subcore_axis_name="s")
    k = functools.partial(
        pl.kernel,
        out_type=jax.ShapeDtypeStruct((8, EMBED_DIM, _NPAD), jnp.float32),
        mesh=mesh,
        compiler_params=pltpu.CompilerParams(needs_layout_passes=False),
        scratch_types=[
            pltpu.VMEM((_PAIRS_MAX,), jnp.int32),
            pltpu.VMEM((_PAIRS, 2 * EMBED_DIM), jnp.float32),
            pltpu.VMEM((_PAIRS, 2 * EMBED_DIM), jnp.float32),
            pltpu.VMEM((EMBED_DIM, _TPITCH), jnp.float32),
            pltpu.VMEM((EMBED_DIM, _TPITCH), jnp.float32),
            pltpu.SemaphoreType.DMA,
            pltpu.SemaphoreType.DMA,
            pltpu.SemaphoreType.DMA,
            pltpu.SemaphoreType.DMA,
        ],
    )(_sc_embed)
    return k(idx2, tab_pair)


def kernel(bin_index, table):
    idx_flat = bin_index.reshape(-1).astype(jnp.int32)
    ipair = idx_flat.reshape(-1, 2)
    idx2 = ipair[:, 0] * NUM_BINS + ipair[:, 1]
    tab_pair = jnp.concatenate(
        [jnp.repeat(table, NUM_BINS, axis=0),
         jnp.tile(table, (NUM_BINS, 1))], axis=1)
    out_t = _run(idx2, tab_pair)
    return jnp.transpose(out_t[:, :, :_N], (0, 2, 1))


# speedup vs baseline: 2.5329x; 2.5329x over previous
import functools

import jax
import jax.numpy as jnp
from jax import lax
from jax.experimental import pallas as pl
from jax.experimental.pallas import tpu as pltpu
from jax.experimental.pallas import tpu_sc as plsc

NUM_BINS = 64
EMBED_DIM = 64
_TAB_W = 128

_NC = 2
_NS = 16
_NW = _NC * _NS

_N = 100000
_NPAD = 100096
_W_PER_ROW = 4
_STRIPE = 25088
_CHUNK = 896
_NFULL = 27
_LAST = _STRIPE - _NFULL * _CHUNK
_TAIL = _NPAD - 3 * _STRIPE - _NFULL * _CHUNK
_TAIL_REAL = _N - 3 * _STRIPE - _NFULL * _CHUNK


def _gather_group(tab, idx_ref, buf, off):
    addr = idx_ref[pl.ds(off, 16)]
    for d in range(EMBED_DIM):
        buf[d, pl.ds(off, 16)] = plsc.load_gather(tab, [addr])
        if d + 1 < EMBED_DIM:
            addr = addr + _TAB_W


def _compute_chunk(tab, idx_ref, buf, n):
    def ngroup(g, _):
        _gather_group(tab, idx_ref, buf, g * 16)
        return 0

    lax.fori_loop(0, n // 16, ngroup, 0)


def _sc_embed(idx_hbm, tab_hbm, out_hbm,
              tab_v, idx0, idx1, buf0, buf1,
              sem_t, sem_i0, sem_i1, sem_o0, sem_o1):
    c = lax.axis_index("c")
    s = lax.axis_index("s")
    wid = s * _NC + c
    out_b = wid // _W_PER_ROW
    t = wid % _W_PER_ROW
    out_n0 = t * _STRIPE
    base = out_b * _N + out_n0

    idx = (idx0, idx1)
    buf = (buf0, buf1)
    sem_i = (sem_i0, sem_i1)
    sem_o = (sem_o0, sem_o1)

    tload = pltpu.async_copy(tab_hbm, tab_v, sem_t)
    loads = [None, None]
    stores = [None, None]
    for i in range(2):
        loads[i] = pltpu.async_copy(
            idx_hbm.at[pl.ds(base + i * _CHUNK, _CHUNK)], idx[i], sem_i[i])
    tload.wait()

    for i in range(_NFULL):
        sl = i % 2
        loads[sl].wait()
        if stores[sl] is not None:
            stores[sl].wait()
        _compute_chunk(tab_v, idx[sl], buf[sl], _CHUNK)
        stores[sl] = pltpu.async_copy(
            buf[sl],
            out_hbm.at[out_b, :, pl.ds(out_n0 + i * _CHUNK, _CHUNK)],
            sem_o[sl])
        if i + 2 < _NFULL:
            loads[sl] = pltpu.async_copy(
                idx_hbm.at[pl.ds(base + (i + 2) * _CHUNK, _CHUNK)],
                idx[sl], sem_i[sl])

    for sl in range(2):
        stores[sl].wait()

    tail_off = _NFULL * _CHUNK

    @pl.when(t < _W_PER_ROW - 1)
    def _():
        pltpu.async_copy(
            idx_hbm.at[pl.ds(base + tail_off, _LAST)], idx0, sem_i0).wait()
        _compute_chunk(tab_v, idx0, buf0, _LAST)
        pltpu.async_copy(
            buf0,
            out_hbm.at[out_b, :, pl.ds(out_n0 + tail_off, _LAST)],
            sem_o0).wait()

    @pl.when(t == _W_PER_ROW - 1)
    def _():
        pltpu.async_copy(
            idx_hbm.at[pl.ds(base + tail_off, _TAIL_REAL)],
            idx1.at[pl.ds(0, _TAIL_REAL)], sem_i1).wait()
        zeros = jnp.zeros((16,), jnp.int32)
        for z in range(_TAIL_REAL, _TAIL, 16):
            idx1[pl.ds(z, 16)] = zeros
        _compute_chunk(tab_v, idx1, buf1, _TAIL)
        pltpu.async_copy(
            buf1.at[:, pl.ds(0, _TAIL)],
            out_hbm.at[out_b, :, pl.ds(out_n0 + tail_off, _TAIL)],
            sem_o1).wait()


@jax.jit
def _run(idx_flat, tab_t):
    mesh = plsc.VectorSubcoreMesh(core_axis_name="c", subcore_axis_name="s")
    k = functools.partial(
        pl.kernel,
        out_type=jax.ShapeDtypeStruct((8, EMBED_DIM, _NPAD), jnp.float32),
        mesh=mesh,
        compiler_params=pltpu.CompilerParams(needs_layout_passes=False),
        scratch_types=[
            pltpu.VMEM((EMBED_DIM * _TAB_W,), jnp.float32),
            pltpu.VMEM((_CHUNK,), jnp.int32),
            pltpu.VMEM((_CHUNK,), jnp.int32),
            pltpu.VMEM((EMBED_DIM, _CHUNK), jnp.float32),
            pltpu.VMEM((EMBED_DIM, _CHUNK), jnp.float32),
            pltpu.SemaphoreType.DMA,
            pltpu.SemaphoreType.DMA,
            pltpu.SemaphoreType.DMA,
            pltpu.SemaphoreType.DMA,
            pltpu.SemaphoreType.DMA,
        ],
    )(_sc_embed)
    return k(idx_flat, tab_t)


def kernel(bin_index, table):
    idx_flat = bin_index.reshape(-1).astype(jnp.int32)
    tab_t = jnp.pad(table.T, ((0, 0), (0, _TAB_W - NUM_BINS))).reshape(-1)
    out_t = _run(idx_flat, tab_t)
    return jnp.transpose(out_t[:, :, :_N], (0, 2, 1))
